# baseline (speedup 1.0000x reference)
import functools

import jax
import jax.numpy as jnp
from jax import lax
from jax.experimental import pallas as pl
from jax.experimental.pallas import tpu as pltpu
from jax.experimental.pallas import tpu_sc as plsc

NC, NS, L = 2, 16, 16
NW = NC * NS


def _slice_offsets(S):
    offs, o = [], 0
    while o + L <= S:
        if (o // 128) != ((o + L - 1) // 128):
            o = ((o // 128) + 1) * 128
        offs.append(o)
        o += L
    if o < S:
        offs.append(S - L)
    return offs


@functools.lru_cache(maxsize=None)
def _gather_fn(B, S, D, V):
    n_b = B // NW
    assert B == NW * n_b and D == 64 and (V - 1) % 2 == 0
    VP = (V - 1) // 2
    fix_offs = _slice_offsets(S)
    n_grp = (S + L - 1) // L
    splits = []
    off = 0
    while off < S:
        g = min(128, S - off)
        splits.append((off, g))
        off += g
    mesh = plsc.VectorSubcoreMesh(core_axis_name="c", subcore_axis_name="s")

    @functools.partial(
        pl.kernel,
        mesh=mesh,
        compiler_params=pltpu.CompilerParams(use_tc_tiling_on_sc=True,
                                             needs_layout_passes=False),
        out_type=jax.ShapeDtypeStruct((B, S, D), jnp.float32),
        scratch_types=[
            pltpu.VMEM((n_b, S), jnp.int32),
            pltpu.VMEM((n_b, S), jnp.int32),
            pltpu.VMEM((S, 2 * D), jnp.float32),
            pltpu.VMEM((1, S, D), jnp.float32),
            pltpu.VMEM((1, D), jnp.float32),
            pltpu.VMEM((L,), jnp.int32),
            pltpu.SemaphoreType.DMA,
        ],
    )
    def gather_kernel(tp_hbm, tl_hbm, idx_hbm, seqlen_hbm, out_hbm,
                      idx_v, half_v, pair_v, rows_v, last_v, seql_v, sem):
        wid = lax.axis_index("s") * NC + lax.axis_index("c")
        b0 = wid * n_b
        pltpu.sync_copy(idx_hbm.at[pl.ds(b0, n_b)], idx_v)
        pltpu.sync_copy(seqlen_hbm, seql_v)
        pltpu.sync_copy(tl_hbm, last_v)
        seql = seql_v[...]
        lane = lax.iota(jnp.int32, L)
        zeros = jnp.zeros((L,), jnp.int32)

        def fix_row(i, carry):
            vals = [idx_v[i, pl.ds(o, L)] for o in fix_offs]
            for o, v in zip(fix_offs, vals):
                col = o + lane
                t = jnp.where(col < seql, v + 1, 0)
                m2 = t == (V - 1)
                p = jnp.where(m2, 0, t >> 1)
                h = jnp.where(m2, 2, t & 1)
                idx_v[i, pl.ds(o, L)] = p
                half_v[i, pl.ds(o, L)] = h
            return carry

        lax.fori_loop(0, n_b, fix_row, 0)

        def chunk(c, carry):
            handles = [
                pltpu.async_copy(tp_hbm.at[idx_v.at[c, pl.ds(off, g)]],
                                 pair_v.at[pl.ds(off, g)], sem)
                for (off, g) in splits
            ]
            for h in handles:
                h.wait()
            c_vec = zeros + c

            def grp(g, carry2):
                s_vec = g * L + lane
                m = s_vec < S
                s_safe = jnp.where(m, s_vec, 0)
                h_vec = plsc.load_gather(half_v, [c_vec, s_safe], mask=m)
                hc64 = jnp.minimum(h_vec, 1) * D
                m2 = h_vec == 2

                def kstep(k, carry3):
                    for kk in range(4):
                        k_vec = zeros + (k * 4 + kk)
                        gv = plsc.load_gather(pair_v, [s_safe, hc64 + k_vec],
                                              mask=m)
                        plsc.store_scatter(rows_v, [zeros, s_safe, k_vec], gv,
                                           mask=m)
                    return carry3

                lax.fori_loop(0, D // 4, kstep, 0)

                n2 = plsc.all_reduce_population_count(m2 & m)

                @pl.when(n2[0] > 0)
                def _fixup():
                    def kfix(k, carry4):
                        k_vec = zeros + k
                        gv = plsc.load_gather(last_v, [zeros, k_vec])
                        plsc.store_scatter(rows_v, [zeros, s_safe, k_vec], gv,
                                           mask=m2 & m)
                        return carry4
                    lax.fori_loop(0, D, kfix, 0)

                return carry2

            lax.fori_loop(0, n_grp, grp, 0)
            pltpu.sync_copy(rows_v, out_hbm.at[pl.ds(b0 + c, 1)])
            return carry

        lax.fori_loop(0, n_b, chunk, 0)

    return gather_kernel


def kernel(text, seq_len, text_embed_weight):
    B, S = text.shape
    V, D = text_embed_weight.shape
    tp = text_embed_weight[:V - 1].reshape((V - 1) // 2, 2 * D)
    tl = text_embed_weight[V - 1:]
    seql_vec = jnp.full((L,), seq_len, dtype=jnp.int32)
    return _gather_fn(B, S, D, V)(tp, tl, text, seql_vec)


# speedup vs baseline: 1.6650x; 1.6650x over previous
import functools

import jax
import jax.numpy as jnp
from jax import lax
from jax.experimental import pallas as pl
from jax.experimental.pallas import tpu as pltpu
from jax.experimental.pallas import tpu_sc as plsc

NC, NS, L = 2, 16, 16
NW = NC * NS

ROWS_PER_CHUNK = 4


@functools.lru_cache(maxsize=None)
def _gather_fn(B, S, D):
    n_b = B // NW
    n_chunks = n_b // ROWS_PER_CHUNK
    n_flat = n_b * S
    assert B == NW * n_b and n_b == n_chunks * ROWS_PER_CHUNK
    assert n_flat % L == 0 and (S % 8) == 0
    splits = []
    off = 0
    while off < S:
        g = min(128, S - off)
        splits.append((off, g))
        off += g
    mesh = plsc.VectorSubcoreMesh(core_axis_name="c", subcore_axis_name="s")

    @functools.partial(
        pl.kernel,
        mesh=mesh,
        compiler_params=pltpu.CompilerParams(use_tc_tiling_on_sc=False),
        out_type=jax.ShapeDtypeStruct((B, S, D), jnp.float32),
        scratch_types=[
            pltpu.VMEM((n_flat,), jnp.int32),
            pltpu.VMEM((ROWS_PER_CHUNK, S, 2 * D), jnp.float32),
            pltpu.VMEM((L,), jnp.int32),
            pltpu.SemaphoreType.DMA,
            pltpu.SemaphoreType.DMA,
        ],
    )
    def gather_kernel(table_hbm, idx_hbm, seqlen_hbm, out_hbm,
                      idx_v, rows_v, seql_v, sem, sem2):
        wid = lax.axis_index("s") * NC + lax.axis_index("c")
        b0 = wid * n_b
        stage = [pltpu.async_copy(idx_hbm.at[b0 + i],
                                  idx_v.at[pl.ds(i * S, S)], sem2)
                 for i in range(n_b)]
        pltpu.sync_copy(seqlen_hbm, seql_v)
        for h in stage:
            h.wait()
        seql = seql_v[...]
        lane = lax.iota(jnp.int32, L)

        def fix(k, carry):
            v = idx_v[pl.ds(k * L, L)]
            col = lax.rem(k * L + lane, S)
            idx_v[pl.ds(k * L, L)] = jnp.where(col < seql, v + 1,
                                               jnp.zeros_like(v))
            return carry

        lax.fori_loop(0, n_flat // L, fix, 0)

        for c in range(n_chunks):
            handles = []
            for i in range(ROWS_PER_CHUNK):
                flat0 = (c * ROWS_PER_CHUNK + i) * S
                for (off, g) in splits:
                    handles.append(pltpu.async_copy(
                        table_hbm.at[idx_v.at[pl.ds(flat0 + off, g)]],
                        rows_v.at[i, pl.ds(off, g)], sem))
            for h in handles:
                h.wait()
            pltpu.sync_copy(
                rows_v.at[:, :, pl.ds(0, D)],
                out_hbm.at[pl.ds(b0 + c * ROWS_PER_CHUNK, ROWS_PER_CHUNK)])

    return gather_kernel


def kernel(text, seq_len, text_embed_weight):
    B, S = text.shape
    D = text_embed_weight.shape[1]
    tpad = jnp.pad(text_embed_weight, ((0, 7), (0, D)))
    seql_vec = jnp.full((L,), seq_len, dtype=jnp.int32)
    return _gather_fn(B, S, D)(tpad, text, seql_vec)
